# baseline (speedup 1.0000x reference)
import functools

import jax
import jax.numpy as jnp
from jax import lax
from jax.experimental import pallas as pl
from jax.experimental.pallas import tpu as pltpu
from jax.experimental.pallas import tpu_sc as plsc

_EPS = 1e-12
_NC = 2
_NS = 16
_NW = _NC * _NS
_CH = 128
_L = 16


def _rsqrt_sc(x):
    i = lax.bitcast_convert_type(x, jnp.int32)
    i = jnp.int32(0x5F3759DF) - lax.shift_right_logical(i, 1)
    y = lax.bitcast_convert_type(i, jnp.float32)
    for _ in range(3):
        y = y * (1.5 - 0.5 * x * y * y)
    return y


def kernel(input_ids, token_type_ids, word_emb, tok_type_emb, ln_weight, ln_bias):
    B, L = input_ids.shape
    V, H = word_emb.shape
    N = B * L
    ids = input_ids.reshape(N).astype(jnp.int32)
    tts = token_type_ids.reshape(N).astype(jnp.int32)

    per_w = N // _NW
    n_chunks = per_w // _CH
    nj = H // _L

    mesh = plsc.VectorSubcoreMesh(core_axis_name="c", subcore_axis_name="s")

    @functools.partial(
        pl.kernel,
        mesh=mesh,
        compiler_params=pltpu.CompilerParams(needs_layout_passes=False),
        out_type=jax.ShapeDtypeStruct((N, H), jnp.float32),
        scratch_types=[
            pltpu.VMEM((2, _CH), jnp.int32),
            pltpu.VMEM((2, _CH + _L), jnp.int32),
            pltpu.VMEM((2, _CH, H), jnp.float32),
            pltpu.VMEM((2, _CH, H), jnp.float32),
            pltpu.VMEM((2, _CH, H), jnp.float32),
            pltpu.VMEM((H,), jnp.float32),
            pltpu.VMEM((H,), jnp.float32),
            pltpu.VMEM((3, H), jnp.float32),
            pltpu.SemaphoreType.DMA,
            pltpu.SemaphoreType.DMA,
            pltpu.SemaphoreType.DMA,
            pltpu.SemaphoreType.DMA,
        ],
    )
    def body(ids_h, tts_h, wemb_h, ttab_h, w_h, b_h, out_h,
             idx_v, tt_v, rows_v, res_v, res2_v, w_v, b_v, ttab_v,
             semw0, semw1, semo0, semo1):
        wid = lax.axis_index("s") * _NC + lax.axis_index("c")
        base = wid * per_w
        pltpu.sync_copy(w_h, w_v)
        pltpu.sync_copy(b_h, b_v)
        pltpu.sync_copy(ttab_h, ttab_v)
        inv_h = jnp.float32(1.0 / H)
        semw = [semw0, semw1]
        semo = [semo0, semo1]

        def start_gathers(c, b):
            start = base + c * _CH
            pltpu.sync_copy(ids_h.at[pl.ds(start, _CH)], idx_v.at[b])
            pltpu.sync_copy(tts_h.at[pl.ds(start, _CH)], tt_v.at[b, pl.ds(0, _CH)])
            pltpu.async_copy(wemb_h.at[idx_v.at[b]], rows_v.at[b], semw[b])

        def wait_gathers(b):
            pltpu.make_async_copy(wemb_h.at[idx_v.at[b]], rows_v.at[b], semw[b]).wait()

        iota = lax.iota(jnp.int32, _L)
        ridx = [iota + g * _L for g in range(_CH // _L)]
        zero16 = jnp.zeros((_L,), jnp.float32)

        def _tree(xs):
            while len(xs) > 1:
                xs = [a + c for a, c in zip(xs[::2], xs[1::2])]
            return xs[0]

        def compute_chunk(b):
            rows = rows_v.at[b]
            res = res_v.at[b]
            res2 = res2_v.at[b]
            ngr = _CH // _L
            du = 8

            stats = []
            for g in range(ngr):
                tvec = tt_v[b, pl.ds(g * _L, _L)]
                rbase = ridx[g]

                def p1(dd, carry, tvec=tvec, rbase=rbase):
                    s, sq = carry
                    base = dd * du
                    xs = []
                    for u in range(du):
                        dmix = (base + u + iota * 8) & (H - 1)
                        x = (plsc.load_gather(rows, [rbase, dmix])
                             + plsc.load_gather(ttab_v, [tvec, dmix]))
                        plsc.store_scatter(res, [rbase, dmix], x)
                        xs.append(x)
                    s = s + _tree(xs)
                    sq = sq + _tree([x * x for x in xs])
                    return s, sq

                s, sq = lax.fori_loop(0, H // du, p1, (zero16, zero16))
                mean = s * inv_h
                var = jnp.maximum(sq * inv_h - mean * mean, 0.0)
                rstd = _rsqrt_sc(var + _EPS)
                stats.append((rstd, -mean * rstd))

            for h in range(ngr // 4):
                gs = list(range(h * 4, h * 4 + 4))

                def p2(dd, carry, gs=gs):
                    for u in range(2):
                        dmix = (dd * 2 + u + iota * 8) & (H - 1)
                        wv = plsc.load_gather(w_v, [dmix])
                        bv = plsc.load_gather(b_v, [dmix])
                        for g in gs:
                            x = plsc.load_gather(res, [ridx[g], dmix])
                            y = (x * stats[g][0] + stats[g][1]) * wv + bv
                            plsc.store_scatter(res2, [ridx[g], dmix], y)
                    return carry

                lax.fori_loop(0, H // 2, p2, 0)

        start_gathers(0, 0)

        def outer(cc, carry):
            for b in range(2):
                c = cc * 2 + b
                @pl.when(c + 1 < n_chunks)
                def _():
                    nb = 1 - b
                    @pl.when(c >= 1)
                    def _():
                        pltpu.make_async_copy(
                            res2_v.at[nb],
                            out_h.at[pl.ds(base + (c - 1) * _CH, _CH)],
                            semo[nb],
                        ).wait()
                    start_gathers(c + 1, nb)

                wait_gathers(b)
                compute_chunk(b)
                pltpu.async_copy(
                    res2_v.at[b], out_h.at[pl.ds(base + c * _CH, _CH)], semo[b]
                )
            return carry

        lax.fori_loop(0, n_chunks // 2, outer, 0)
        for b in range(2):
            c = n_chunks - 2 + b
            pltpu.make_async_copy(
                res2_v.at[b], out_h.at[pl.ds(base + c * _CH, _CH)], semo[b]
            ).wait()

    out = body(ids, tts, word_emb, tok_type_emb, ln_weight, ln_bias)
    return out.reshape(B, L, H)


# speedup vs baseline: 1.8398x; 1.8398x over previous
import functools

import jax
import jax.numpy as jnp
from jax import lax
from jax.experimental import pallas as pl
from jax.experimental.pallas import tpu as pltpu
from jax.experimental.pallas import tpu_sc as plsc

_EPS = 1e-12
_NC = 2
_NS = 16
_NW = _NC * _NS
_CH = 128
_L = 16


def _rsqrt_sc(x):
    i = lax.bitcast_convert_type(x, jnp.int32)
    i = jnp.int32(0x5F3759DF) - lax.shift_right_logical(i, 1)
    y = lax.bitcast_convert_type(i, jnp.float32)
    for _ in range(3):
        y = y * (1.5 - 0.5 * x * y * y)
    return y


def kernel(input_ids, token_type_ids, word_emb, tok_type_emb, ln_weight, ln_bias):
    B, L = input_ids.shape
    V, H = word_emb.shape
    N = B * L
    ids = input_ids.reshape(N).astype(jnp.int32)
    tts = token_type_ids.reshape(N).astype(jnp.int32)

    per_w = N // _NW
    n_chunks = per_w // _CH
    nj = H // _L

    mesh = plsc.VectorSubcoreMesh(core_axis_name="c", subcore_axis_name="s")

    @functools.partial(
        pl.kernel,
        mesh=mesh,
        compiler_params=pltpu.CompilerParams(needs_layout_passes=False),
        out_type=jax.ShapeDtypeStruct((N, H), jnp.float32),
        scratch_types=[
            pltpu.VMEM((2, _CH), jnp.int32),
            pltpu.VMEM((2, _CH + _L), jnp.int32),
            pltpu.VMEM((2, _CH, H), jnp.float32),
            pltpu.VMEM((2, _CH, H), jnp.float32),
            pltpu.VMEM((2, _CH, H), jnp.float32),
            pltpu.VMEM((H,), jnp.float32),
            pltpu.VMEM((H,), jnp.float32),
            pltpu.VMEM((3, H), jnp.float32),
            pltpu.SemaphoreType.DMA,
            pltpu.SemaphoreType.DMA,
            pltpu.SemaphoreType.DMA,
            pltpu.SemaphoreType.DMA,
        ],
    )
    def body(ids_h, tts_h, wemb_h, ttab_h, w_h, b_h, out_h,
             idx_v, tt_v, rows_v, res_v, res2_v, w_v, b_v, ttab_v,
             semw0, semw1, semo0, semo1):
        wid = lax.axis_index("s") * _NC + lax.axis_index("c")
        base = wid * per_w
        pltpu.sync_copy(w_h, w_v)
        pltpu.sync_copy(b_h, b_v)
        pltpu.sync_copy(ttab_h, ttab_v)
        inv_h = jnp.float32(1.0 / H)
        semw = [semw0, semw1]
        semo = [semo0, semo1]

        def start_gathers(c, b):
            start = base + c * _CH
            pltpu.sync_copy(ids_h.at[pl.ds(start, _CH)], idx_v.at[b])
            pltpu.sync_copy(tts_h.at[pl.ds(start, _CH)], tt_v.at[b, pl.ds(0, _CH)])
            pltpu.async_copy(wemb_h.at[idx_v.at[b]], rows_v.at[b], semw[b])

        def wait_gathers(b):
            pltpu.make_async_copy(wemb_h.at[idx_v.at[b]], rows_v.at[b], semw[b]).wait()

        iota = lax.iota(jnp.int32, _L)
        ridx = [iota + g * _L for g in range(_CH // _L)]
        zero16 = jnp.zeros((_L,), jnp.float32)

        def _tree(xs):
            while len(xs) > 1:
                xs = [a + c for a, c in zip(xs[::2], xs[1::2])]
            return xs[0]

        Wj = [w_v[pl.ds(j * _L, _L)] for j in range(nj)]
        Bj = [b_v[pl.ds(j * _L, _L)] for j in range(nj)]

        def compute_chunk(b):
            rows = rows_v.at[b]
            res = res_v.at[b]
            res2 = res2_v.at[b]

            def stageA(r):
                t16 = tt_v[b, pl.ds(r, _L)]
                t_sc = t16[0]
                xs = []
                for j in range(nj):
                    xj = rows[r, pl.ds(j * _L, _L)] + ttab_v[t_sc, pl.ds(j * _L, _L)]
                    res[r, pl.ds(j * _L, _L)] = xj
                    xs.append(xj)
                return _tree(xs), _tree([x * x for x in xs])

            def stageB(ssq):
                s, sq = ssq
                mean = jnp.sum(s) * inv_h
                var = jnp.maximum(jnp.sum(sq) * inv_h - mean * mean, 0.0)
                return mean, _rsqrt_sc(var + _EPS)

            def stageC(r, st):
                mean, rstd = st
                for j in range(nj):
                    aj = rstd * Wj[j]
                    cj = Bj[j] - mean * aj
                    res2[r, pl.ds(j * _L, _L)] = res[r, pl.ds(j * _L, _L)] * aj + cj

            c0 = stageA(0)
            st0 = stageB(c0)
            c1 = stageA(1)

            def main(r, carry):
                sprev, stprev = carry
                stageC(r - 2, stprev)
                snew = stageA(r)
                stnew = stageB(sprev)
                return snew, stnew

            sl, stl = lax.fori_loop(2, _CH, main, (c1, st0), unroll=2)
            stageC(_CH - 2, stl)
            stageC(_CH - 1, stageB(sl))

        start_gathers(0, 0)

        def outer(cc, carry):
            for b in range(2):
                c = cc * 2 + b
                @pl.when(c + 1 < n_chunks)
                def _():
                    nb = 1 - b
                    @pl.when(c >= 1)
                    def _():
                        pltpu.make_async_copy(
                            res2_v.at[nb],
                            out_h.at[pl.ds(base + (c - 1) * _CH, _CH)],
                            semo[nb],
                        ).wait()
                    start_gathers(c + 1, nb)

                wait_gathers(b)
                compute_chunk(b)
                pltpu.async_copy(
                    res2_v.at[b], out_h.at[pl.ds(base + c * _CH, _CH)], semo[b]
                )
            return carry

        lax.fori_loop(0, n_chunks // 2, outer, 0)
        for b in range(2):
            c = n_chunks - 2 + b
            pltpu.make_async_copy(
                res2_v.at[b], out_h.at[pl.ds(base + c * _CH, _CH)], semo[b]
            ).wait()

    out = body(ids, tts, word_emb, tok_type_emb, ln_weight, ln_bias)
    return out.reshape(B, L, H)
